# baseline (speedup 1.0000x reference)
import functools

import jax
import jax.numpy as jnp
from jax import lax
from jax.experimental import pallas as pl
from jax.experimental.pallas import tpu as pltpu
from jax.experimental.pallas import tpu_sc as plsc

N_OBJ = 10000
N_EVT = 50000
E = 320000
D = 128

NC = 2
NS = 16
NW = NC * NS
CHUNK = 64
NCHUNK = 160
NBUF = 4
NPASS = 4
HCHUNK = NCHUNK // NPASS
E_PER = NCHUNK * CHUNK
E_PAD = NW * E_PER
N_OBJ_PAD = 10240
ROWS_PER_SUB = N_OBJ_PAD // NS
ZROWS = 64


def _proj_body(ev_ref, w_ref, b_ref, out_ref):
    x = ev_ref[...]
    acc = jnp.dot(x, w_ref[...], preferred_element_type=jnp.float32)
    out_ref[...] = jnp.maximum(acc + b_ref[...], 0.0)


def _project_events(event_X, W_projT, b_proj2d):
    blk = 2000
    grid = N_EVT // blk
    return pl.pallas_call(
        _proj_body,
        grid=(grid,),
        in_specs=[
            pl.BlockSpec((blk, D), lambda i: (i, 0)),
            pl.BlockSpec((D, D), lambda i: (0, 0)),
            pl.BlockSpec((1, D), lambda i: (0, 0)),
        ],
        out_specs=pl.BlockSpec((blk, D), lambda i: (i, 0)),
        out_shape=jax.ShapeDtypeStruct((N_EVT, D), jnp.float32),
    )(event_X, W_projT, b_proj2d)


def _seg_body(evt_idx_hbm, obj_idx_hbm, p_hbm, sums_out, counts_out,
              evt_v, obj_v, rows0, rows1, rows2, rows3, ones_v, czero,
              gsem0, gsem1, gsem2, gsem3, ssem0, ssem1, ssem2, ssem3,
              csem0, csem1, csem2, csem3, shared_sums, shared_counts):
    c = lax.axis_index("c")
    s = lax.axis_index("s")
    wid = c * NS + s
    rows = [rows0, rows1, rows2, rows3]
    gsems = [gsem0, gsem1, gsem2, gsem3]
    ssems = [ssem0, ssem1, ssem2, ssem3]
    csems = [csem0, csem1, csem2, csem3]

    def fill_ones(i, _):
        ones_v[pl.ds(i * 16, 16)] = jnp.ones((16,), jnp.float32)
        return 0
    lax.fori_loop(0, CHUNK // 16, fill_ones, 0)

    def fill_zero_c(i, _):
        czero[pl.ds(i * 16, 16)] = jnp.zeros((16,), jnp.float32)
        return 0
    lax.fori_loop(0, ROWS_PER_SUB // 16, fill_zero_c, 0)

    def fill_zero(i, _):
        rows0[i // 8, pl.ds((i % 8) * 16, 16)] = jnp.zeros((16,), jnp.float32)
        return 0
    lax.fori_loop(0, ZROWS * 8, fill_zero, 0)

    def zero_sums(k, _):
        pltpu.sync_copy(
            rows0.at[pl.ds(0, ZROWS)],
            shared_sums.at[pl.ds(s * ROWS_PER_SUB + k * ZROWS, ZROWS)])
        return 0
    lax.fori_loop(0, ROWS_PER_SUB // ZROWS, zero_sums, 0)

    pltpu.sync_copy(czero, shared_counts.at[pl.ds(s * ROWS_PER_SUB, ROWS_PER_SUB)])

    plsc.subcore_barrier()

    DLY = 2

    def emit_chunk(jb, b, do_prev):
        pltpu.make_async_copy(p_hbm.at[evt_v.at[jb]], rows[b], gsems[b]).wait()
        pltpu.async_copy(rows[b], shared_sums.at[obj_v.at[jb]], ssems[b],
                         add=True)
        pltpu.async_copy(ones_v, shared_counts.at[obj_v.at[jb]], csems[b],
                         add=True)
        if do_prev:
            pb = (b - DLY) % NBUF
            jp = jb - DLY
            pltpu.make_async_copy(rows[pb], shared_sums.at[obj_v.at[jp]],
                                  ssems[pb]).wait()
            pltpu.make_async_copy(ones_v, shared_counts.at[obj_v.at[jp]],
                                  csems[pb]).wait()
            pltpu.async_copy(
                p_hbm.at[evt_v.at[lax.rem(jp + NBUF, HCHUNK)]],
                rows[pb], gsems[pb])

    for p in range(NPASS):
        row = wid * NPASS + p
        pltpu.sync_copy(evt_idx_hbm.at[row], evt_v)
        pltpu.sync_copy(obj_idx_hbm.at[row], obj_v)

        for b in range(NBUF):
            pltpu.async_copy(p_hbm.at[evt_v.at[b]], rows[b], gsems[b])

        for b in range(NBUF):
            emit_chunk(b, b, b >= DLY)

        def chunk_body(jj, _):
            j = jj * NBUF
            for b in range(NBUF):
                emit_chunk(j + b, b, True)
            return 0
        lax.fori_loop(1, HCHUNK // NBUF, chunk_body, 0)

        for d in range(DLY):
            jl = HCHUNK - DLY + d
            bl = NBUF - DLY + d
            pltpu.make_async_copy(rows[bl], shared_sums.at[obj_v.at[jl]],
                                  ssems[bl]).wait()
            pltpu.make_async_copy(ones_v, shared_counts.at[obj_v.at[jl]],
                                  csems[bl]).wait()
        for b in range(NBUF - DLY):
            pltpu.make_async_copy(p_hbm.at[evt_v.at[b]], rows[b],
                                  gsems[b]).wait()

    plsc.subcore_barrier()

    pltpu.sync_copy(shared_sums.at[pl.ds(s * ROWS_PER_SUB, ROWS_PER_SUB)],
                    sums_out.at[c, pl.ds(s * ROWS_PER_SUB, ROWS_PER_SUB)])
    pltpu.sync_copy(shared_counts.at[pl.ds(s * ROWS_PER_SUB, ROWS_PER_SUB)],
                    counts_out.at[c, pl.ds(s * ROWS_PER_SUB, ROWS_PER_SUB)])


def _segment_mean_parts(evt_idx, obj_idx, P):
    seg = pl.kernel(
        _seg_body,
        out_type=[
            jax.ShapeDtypeStruct((NC, N_OBJ_PAD, D), jnp.float32),
            jax.ShapeDtypeStruct((NC, N_OBJ_PAD), jnp.float32),
        ],
        mesh=plsc.VectorSubcoreMesh(core_axis_name="c", subcore_axis_name="s"),
        scratch_types=[
            pltpu.VMEM((HCHUNK, CHUNK), jnp.int32),
            pltpu.VMEM((HCHUNK, CHUNK), jnp.int32),
            pltpu.VMEM((CHUNK, D), jnp.float32),
            pltpu.VMEM((CHUNK, D), jnp.float32),
            pltpu.VMEM((CHUNK, D), jnp.float32),
            pltpu.VMEM((CHUNK, D), jnp.float32),
            pltpu.VMEM((CHUNK,), jnp.float32),
            pltpu.VMEM((ROWS_PER_SUB,), jnp.float32),
            pltpu.SemaphoreType.DMA,
            pltpu.SemaphoreType.DMA,
            pltpu.SemaphoreType.DMA,
            pltpu.SemaphoreType.DMA,
            pltpu.SemaphoreType.DMA,
            pltpu.SemaphoreType.DMA,
            pltpu.SemaphoreType.DMA,
            pltpu.SemaphoreType.DMA,
            pltpu.SemaphoreType.DMA,
            pltpu.SemaphoreType.DMA,
            pltpu.SemaphoreType.DMA,
            pltpu.SemaphoreType.DMA,
            pltpu.VMEM_SHARED((N_OBJ_PAD, D), jnp.float32),
            pltpu.VMEM_SHARED((N_OBJ_PAD,), jnp.float32),
        ],
    )
    return seg(evt_idx, obj_idx, P)


def _gru_body(sums_ref, counts_ref, hx_ref, wih_ref, whh_ref, bih_ref,
              bhh_ref, mask_ref, out_ref):
    sums = sums_ref[0] + sums_ref[1]
    cnt = counts_ref[0] + counts_ref[1]
    mean = sums / jnp.maximum(cnt, 1.0)
    hx = hx_ref[...]
    gi = jnp.dot(mean, wih_ref[...], preferred_element_type=jnp.float32) + bih_ref[...]
    gh = jnp.dot(hx, whh_ref[...], preferred_element_type=jnp.float32) + bhh_ref[...]
    r = jax.nn.sigmoid(gi[:, 0:D] + gh[:, 0:D])
    z = jax.nn.sigmoid(gi[:, D:2 * D] + gh[:, D:2 * D])
    n = jnp.tanh(gi[:, 2 * D:] + r * gh[:, 2 * D:])
    upd = (1.0 - z) * n + z * hx
    out_ref[...] = hx + mask_ref[...] * (upd - hx)


def _gru_update(sums_p, counts_p, object_X, WihT, WhhT, bih2d, bhh2d, mask2d):
    blk = 2000
    grid = N_OBJ // blk
    return pl.pallas_call(
        _gru_body,
        grid=(grid,),
        in_specs=[
            pl.BlockSpec((NC, blk, D), lambda i: (0, i, 0)),
            pl.BlockSpec((NC, blk, 1), lambda i: (0, i, 0)),
            pl.BlockSpec((blk, D), lambda i: (i, 0)),
            pl.BlockSpec((D, 3 * D), lambda i: (0, 0)),
            pl.BlockSpec((D, 3 * D), lambda i: (0, 0)),
            pl.BlockSpec((1, 3 * D), lambda i: (0, 0)),
            pl.BlockSpec((1, 3 * D), lambda i: (0, 0)),
            pl.BlockSpec((blk, 1), lambda i: (i, 0)),
        ],
        out_specs=pl.BlockSpec((blk, D), lambda i: (i, 0)),
        out_shape=jax.ShapeDtypeStruct((N_OBJ, D), jnp.float32),
    )(sums_p, counts_p, object_X, WihT, WhhT, bih2d, bhh2d, mask2d)


def kernel(object_X, event_X, lc_obj_idx, lc_evt_idx, main_object,
           W_proj, b_proj, W_ih, W_hh, b_ih, b_hh):
    pad = E_PAD - E
    spread = jnp.arange(pad, dtype=jnp.int32)
    evt_idx = jnp.concatenate(
        [lc_evt_idx.astype(jnp.int32), spread % N_EVT]
    ).reshape(NW * NPASS, HCHUNK, CHUNK)
    obj_idx = jnp.concatenate(
        [lc_obj_idx.astype(jnp.int32),
         N_OBJ + spread % (N_OBJ_PAD - N_OBJ)]
    ).reshape(NW * NPASS, HCHUNK, CHUNK)

    P = _project_events(event_X, W_proj.T, b_proj.reshape(1, D))
    sums_p, counts_p = _segment_mean_parts(evt_idx, obj_idx, P)
    counts_p = counts_p.reshape(NC, N_OBJ_PAD, 1)

    mask2d = main_object.astype(jnp.float32).reshape(N_OBJ, 1)
    return _gru_update(sums_p, counts_p, object_X,
                       W_ih.T, W_hh.T,
                       b_ih.reshape(1, 3 * D), b_hh.reshape(1, 3 * D),
                       mask2d)


# speedup vs baseline: 1.0594x; 1.0594x over previous
import functools

import jax
import jax.numpy as jnp
from jax import lax
from jax.experimental import pallas as pl
from jax.experimental.pallas import tpu as pltpu
from jax.experimental.pallas import tpu_sc as plsc

N_OBJ = 10000
N_EVT = 50000
E = 320000
D = 128

NC = 2
NS = 16
NW = NC * NS
CHUNK = 64
NCHUNK = 160
NBUF = 4
NPASS = 4
HCHUNK = NCHUNK // NPASS
E_PER = NCHUNK * CHUNK
E_PAD = NW * E_PER
N_OBJ_PAD = 10240
ROWS_PER_SUB = N_OBJ_PAD // NS
ZROWS = 64


def _proj_body(ev_ref, w_ref, b_ref, out_ref):
    x = ev_ref[...]
    acc = jnp.dot(x, w_ref[...], preferred_element_type=jnp.float32)
    out_ref[...] = jnp.maximum(acc + b_ref[...], 0.0)


def _project_events(event_X, W_projT, b_proj2d):
    blk = 2000
    grid = N_EVT // blk
    return pl.pallas_call(
        _proj_body,
        grid=(grid,),
        in_specs=[
            pl.BlockSpec((blk, D), lambda i: (i, 0)),
            pl.BlockSpec((D, D), lambda i: (0, 0)),
            pl.BlockSpec((1, D), lambda i: (0, 0)),
        ],
        out_specs=pl.BlockSpec((blk, D), lambda i: (i, 0)),
        out_shape=jax.ShapeDtypeStruct((N_EVT, D), jnp.float32),
    )(event_X, W_projT, b_proj2d)


def _seg_body(evt_idx_hbm, obj_idx_hbm, p_hbm, sums_out, counts_out,
              evt_v, obj_v, rows0, rows1, rows2, rows3, ones_v, czero,
              gsem0, gsem1, gsem2, gsem3, ssem0, ssem1, ssem2, ssem3,
              csem0, csem1, csem2, csem3, shared_sums, shared_counts):
    c = lax.axis_index("c")
    s = lax.axis_index("s")
    wid = c * NS + s
    rows = [rows0, rows1, rows2, rows3]
    gsems = [gsem0, gsem1, gsem2, gsem3]
    ssems = [ssem0, ssem1, ssem2, ssem3]
    csems = [csem0, csem1, csem2, csem3]

    def fill_ones(i, _):
        ones_v[pl.ds(i * 16, 16)] = jnp.ones((16,), jnp.float32)
        return 0
    lax.fori_loop(0, CHUNK // 16, fill_ones, 0)

    def fill_zero_c(i, _):
        czero[pl.ds(i * 16, 16)] = jnp.zeros((16,), jnp.float32)
        return 0
    lax.fori_loop(0, ROWS_PER_SUB // 16, fill_zero_c, 0)

    def fill_zero(i, _):
        rows0[i // 8, pl.ds((i % 8) * 16, 16)] = jnp.zeros((16,), jnp.float32)
        return 0
    lax.fori_loop(0, ZROWS * 8, fill_zero, 0)

    def zero_sums(k, _):
        pltpu.sync_copy(
            rows0.at[pl.ds(0, ZROWS)],
            shared_sums.at[pl.ds(s * ROWS_PER_SUB + k * ZROWS, ZROWS)])
        return 0
    lax.fori_loop(0, ROWS_PER_SUB // ZROWS, zero_sums, 0)

    pltpu.sync_copy(czero, shared_counts.at[pl.ds(s * ROWS_PER_SUB, ROWS_PER_SUB)])

    plsc.subcore_barrier()

    DLY = 1

    def emit_chunk(jb, b, do_prev):
        pltpu.make_async_copy(p_hbm.at[evt_v.at[jb]], rows[b], gsems[b]).wait()
        pltpu.async_copy(rows[b], shared_sums.at[obj_v.at[jb]], ssems[b],
                         add=True)
        pltpu.async_copy(ones_v, shared_counts.at[obj_v.at[jb]], csems[b],
                         add=True)
        if do_prev:
            pb = (b - DLY) % NBUF
            jp = jb - DLY
            pltpu.make_async_copy(rows[pb], shared_sums.at[obj_v.at[jp]],
                                  ssems[pb]).wait()
            pltpu.make_async_copy(ones_v, shared_counts.at[obj_v.at[jp]],
                                  csems[pb]).wait()
            pltpu.async_copy(
                p_hbm.at[evt_v.at[lax.rem(jp + NBUF, HCHUNK)]],
                rows[pb], gsems[pb])

    for p in range(NPASS):
        row = wid * NPASS + p
        pltpu.sync_copy(evt_idx_hbm.at[row], evt_v)
        pltpu.sync_copy(obj_idx_hbm.at[row], obj_v)

        for b in range(NBUF):
            pltpu.async_copy(p_hbm.at[evt_v.at[b]], rows[b], gsems[b])

        for b in range(NBUF):
            emit_chunk(b, b, b >= DLY)

        def chunk_body(jj, _):
            j = jj * NBUF
            for b in range(NBUF):
                emit_chunk(j + b, b, True)
            return 0
        lax.fori_loop(1, HCHUNK // NBUF, chunk_body, 0)

        for d in range(DLY):
            jl = HCHUNK - DLY + d
            bl = NBUF - DLY + d
            pltpu.make_async_copy(rows[bl], shared_sums.at[obj_v.at[jl]],
                                  ssems[bl]).wait()
            pltpu.make_async_copy(ones_v, shared_counts.at[obj_v.at[jl]],
                                  csems[bl]).wait()
        for b in range(NBUF - DLY):
            pltpu.make_async_copy(p_hbm.at[evt_v.at[b]], rows[b],
                                  gsems[b]).wait()

    plsc.subcore_barrier()

    pltpu.sync_copy(shared_sums.at[pl.ds(s * ROWS_PER_SUB, ROWS_PER_SUB)],
                    sums_out.at[c, pl.ds(s * ROWS_PER_SUB, ROWS_PER_SUB)])
    pltpu.sync_copy(shared_counts.at[pl.ds(s * ROWS_PER_SUB, ROWS_PER_SUB)],
                    counts_out.at[c, pl.ds(s * ROWS_PER_SUB, ROWS_PER_SUB)])


def _segment_mean_parts(evt_idx, obj_idx, P):
    seg = pl.kernel(
        _seg_body,
        out_type=[
            jax.ShapeDtypeStruct((NC, N_OBJ_PAD, D), jnp.float32),
            jax.ShapeDtypeStruct((NC, N_OBJ_PAD), jnp.float32),
        ],
        mesh=plsc.VectorSubcoreMesh(core_axis_name="c", subcore_axis_name="s"),
        scratch_types=[
            pltpu.VMEM((HCHUNK, CHUNK), jnp.int32),
            pltpu.VMEM((HCHUNK, CHUNK), jnp.int32),
            pltpu.VMEM((CHUNK, D), jnp.float32),
            pltpu.VMEM((CHUNK, D), jnp.float32),
            pltpu.VMEM((CHUNK, D), jnp.float32),
            pltpu.VMEM((CHUNK, D), jnp.float32),
            pltpu.VMEM((CHUNK,), jnp.float32),
            pltpu.VMEM((ROWS_PER_SUB,), jnp.float32),
            pltpu.SemaphoreType.DMA,
            pltpu.SemaphoreType.DMA,
            pltpu.SemaphoreType.DMA,
            pltpu.SemaphoreType.DMA,
            pltpu.SemaphoreType.DMA,
            pltpu.SemaphoreType.DMA,
            pltpu.SemaphoreType.DMA,
            pltpu.SemaphoreType.DMA,
            pltpu.SemaphoreType.DMA,
            pltpu.SemaphoreType.DMA,
            pltpu.SemaphoreType.DMA,
            pltpu.SemaphoreType.DMA,
            pltpu.VMEM_SHARED((N_OBJ_PAD, D), jnp.float32),
            pltpu.VMEM_SHARED((N_OBJ_PAD,), jnp.float32),
        ],
    )
    return seg(evt_idx, obj_idx, P)


def _gru_body(sums_ref, counts_ref, hx_ref, wih_ref, whh_ref, bih_ref,
              bhh_ref, mask_ref, out_ref):
    sums = sums_ref[0] + sums_ref[1]
    cnt = counts_ref[0] + counts_ref[1]
    mean = sums / jnp.maximum(cnt, 1.0)
    hx = hx_ref[...]
    gi = jnp.dot(mean, wih_ref[...], preferred_element_type=jnp.float32) + bih_ref[...]
    gh = jnp.dot(hx, whh_ref[...], preferred_element_type=jnp.float32) + bhh_ref[...]
    r = jax.nn.sigmoid(gi[:, 0:D] + gh[:, 0:D])
    z = jax.nn.sigmoid(gi[:, D:2 * D] + gh[:, D:2 * D])
    n = jnp.tanh(gi[:, 2 * D:] + r * gh[:, 2 * D:])
    upd = (1.0 - z) * n + z * hx
    out_ref[...] = hx + mask_ref[...] * (upd - hx)


def _gru_update(sums_p, counts_p, object_X, WihT, WhhT, bih2d, bhh2d, mask2d):
    blk = 2000
    grid = N_OBJ // blk
    return pl.pallas_call(
        _gru_body,
        grid=(grid,),
        in_specs=[
            pl.BlockSpec((NC, blk, D), lambda i: (0, i, 0)),
            pl.BlockSpec((NC, blk, 1), lambda i: (0, i, 0)),
            pl.BlockSpec((blk, D), lambda i: (i, 0)),
            pl.BlockSpec((D, 3 * D), lambda i: (0, 0)),
            pl.BlockSpec((D, 3 * D), lambda i: (0, 0)),
            pl.BlockSpec((1, 3 * D), lambda i: (0, 0)),
            pl.BlockSpec((1, 3 * D), lambda i: (0, 0)),
            pl.BlockSpec((blk, 1), lambda i: (i, 0)),
        ],
        out_specs=pl.BlockSpec((blk, D), lambda i: (i, 0)),
        out_shape=jax.ShapeDtypeStruct((N_OBJ, D), jnp.float32),
    )(sums_p, counts_p, object_X, WihT, WhhT, bih2d, bhh2d, mask2d)


def kernel(object_X, event_X, lc_obj_idx, lc_evt_idx, main_object,
           W_proj, b_proj, W_ih, W_hh, b_ih, b_hh):
    pad = E_PAD - E
    spread = jnp.arange(pad, dtype=jnp.int32)
    evt_idx = jnp.concatenate(
        [lc_evt_idx.astype(jnp.int32), spread % N_EVT]
    ).reshape(NW * NPASS, HCHUNK, CHUNK)
    obj_idx = jnp.concatenate(
        [lc_obj_idx.astype(jnp.int32),
         N_OBJ + spread % (N_OBJ_PAD - N_OBJ)]
    ).reshape(NW * NPASS, HCHUNK, CHUNK)

    P = _project_events(event_X, W_proj.T, b_proj.reshape(1, D))
    sums_p, counts_p = _segment_mean_parts(evt_idx, obj_idx, P)
    counts_p = counts_p.reshape(NC, N_OBJ_PAD, 1)

    mask2d = main_object.astype(jnp.float32).reshape(N_OBJ, 1)
    return _gru_update(sums_p, counts_p, object_X,
                       W_ih.T, W_hh.T,
                       b_ih.reshape(1, 3 * D), b_hh.reshape(1, 3 * D),
                       mask2d)
